# Initial kernel scaffold; baseline (speedup 1.0000x reference)
import functools

import jax
import jax.numpy as jnp
from jax import lax
from jax.experimental import pallas as pl
from jax.experimental.pallas import tpu as pltpu
from jax.experimental.pallas import tpu_sc as plsc

L = 16
GRP = 64
KT = 2048
NC, NS = 2, 16
NW = NC * NS

NEG = jnp.float32(-1e30)
NEGF = jnp.float32(-3e38)
BIGI = jnp.int32(2**30)

CAP_FLUSH = 64
CH = 64


def _tc_body(nk, q_ref, k_ref, sim_ref, bmax_ref):
    j = pl.program_id(0)
    q = q_ref[...]
    qn = q / (jnp.sqrt(jnp.sum(q * q, axis=1, keepdims=True)) + 1e-8)
    k = k_ref[...]
    kn = k / (jnp.sqrt(jnp.sum(k * k, axis=1, keepdims=True)) + 1e-8)
    sim = lax.dot_general(qn, kn, (((1,), (1,)), ((), ())),
                          preferred_element_type=jnp.float32)
    col = j * KT + lax.broadcasted_iota(jnp.int32, sim.shape, 1)
    sim = jnp.where(col < nk, sim, NEG)
    sim_ref[...] = sim
    bmax_ref[...] = jnp.max(
        sim.reshape(sim.shape[0], KT // GRP, GRP), axis=2)


def _tc_sim(queries, keys_p, nk):
    nq, d = queries.shape
    kpad = keys_p.shape[0]
    grid = kpad // KT
    return pl.pallas_call(
        functools.partial(_tc_body, nk),
        grid=(grid,),
        in_specs=[
            pl.BlockSpec((nq, d), lambda j: (0, 0)),
            pl.BlockSpec((KT, d), lambda j: (j, 0)),
        ],
        out_specs=[
            pl.BlockSpec((nq, KT), lambda j: (0, j)),
            pl.BlockSpec((nq, KT // GRP), lambda j: (0, j)),
        ],
        out_shape=[
            jax.ShapeDtypeStruct((nq, kpad), jnp.float32),
            jax.ShapeDtypeStruct((nq, kpad // GRP), jnp.float32),
        ],
    )(queries, keys_p)


def _iota16():
    return lax.iota(jnp.int32, (L,))


def _sortd(v):
    return lax.rev(lax.sort(v), (0,))


def _sortd_kv(k, v):
    ks, vs = plsc.sort_key_val(k, v, descending=True)
    return ks, vs


def _merge_desc(a, b):
    br = lax.rev(b, (0,))
    m = a >= br
    hi = jnp.where(m, a, br)
    lo = jnp.where(m, br, a)
    return _sortd(hi), _sortd(lo)


def _merge32(c, t0, t1):
    hi, lo = _merge_desc(c, t1)
    t0n, b = _merge_desc(hi, t0)
    t1n, _ = _merge_desc(b, lo)
    return t0n, t1n


def _merge_desc_kv(ak, av, bk, bv):
    brk = lax.rev(bk, (0,))
    brv = lax.rev(bv, (0,))
    m = (ak > brk) | ((ak == brk) & (av < brv))
    hik = jnp.where(m, ak, brk)
    hiv = jnp.where(m, av, brv)
    lok = jnp.where(m, brk, ak)
    lov = jnp.where(m, brv, av)
    hik, hiv = _sortd_kv(hik, hiv)
    lok, lov = _sortd_kv(lok, lov)
    return hik, hiv, lok, lov


def _merge32_kv(ck, cv, t0, i0, t1, i1):
    hik, hiv, lok, lov = _merge_desc_kv(ck, cv, t1, i1)
    t0n, i0n, bk, bv = _merge_desc_kv(hik, hiv, t0, i0)
    t1n, i1n, _, _ = _merge_desc_kv(bk, bv, lok, lov)
    return t0n, i0n, t1n, i1n


def _popcount(m):
    return jnp.max(plsc.all_reduce_population_count(m))


def _sc_retrieve(simb, bmax, values):
    nq, nb = bmax.shape
    kn, d = values.shape
    nvec = nb // L
    qpw = nq // NW
    mesh = plsc.VectorSubcoreMesh(core_axis_name="c", subcore_axis_name="s")

    @functools.partial(
        pl.kernel,
        out_type=jax.ShapeDtypeStruct((nq, d), jnp.float32),
        mesh=mesh,
        scratch_types=[
            pltpu.VMEM((nb,), jnp.float32),
            pltpu.VMEM((nb + CH,), jnp.int32),
            pltpu.VMEM((96,), jnp.float32),
            pltpu.VMEM((96,), jnp.int32),
            pltpu.VMEM((CH,), jnp.int32),
            pltpu.VMEM((CH, GRP), jnp.float32),
            pltpu.VMEM((32,), jnp.int32),
            pltpu.VMEM((32,), jnp.float32),
            pltpu.VMEM((32, d), jnp.float32),
            pltpu.VMEM((d,), jnp.float32),
            pltpu.SemaphoreType.DMA,
        ],
    )
    def body(simb_hbm, bmax_hbm, values_hbm, out_hbm,
             bmax_v, bid_v, cand_v, candi_v, gid_v, blocks_v,
             idx_v, w_v, rows_v, out_v, sem):
        wid = lax.axis_index("c") * NS + lax.axis_index("s")

        def flush_vals(t0, t1, cnt):
            nch = (cnt + L - 1) // L

            def fb(ci, c2):
                t0, t1 = c2
                v = cand_v[pl.ds(ci * L, L)]
                v = jnp.where(_iota16() < (cnt - ci * L), v, NEGF)
                return _merge32(_sortd(v), t0, t1)

            return lax.fori_loop(0, nch, fb, (t0, t1))

        def flush_kv(t0, i0, t1, i1, cnt):
            nch = (cnt + L - 1) // L

            def fb(ci, c2):
                t0, i0, t1, i1 = c2
                v = cand_v[pl.ds(ci * L, L)]
                iv = candi_v[pl.ds(ci * L, L)]
                valid = _iota16() < (cnt - ci * L)
                v = jnp.where(valid, v, NEGF)
                iv = jnp.where(valid, iv, BIGI)
                ck, cv = _sortd_kv(v, iv)
                return _merge32_kv(ck, cv, t0, i0, t1, i1)

            return lax.fori_loop(0, nch, fb, (t0, i0, t1, i1))

        def per_query(qi, _):
            q = wid * qpw + qi
            pltpu.sync_copy(bmax_hbm.at[q], bmax_v)

            v0 = _sortd(bmax_v[pl.ds(0, L)])
            v1 = _sortd(bmax_v[pl.ds(L, L)])
            t0, t1 = _merge_desc(v0, v1)
            thr = jnp.min(t1)

            def body_a(j, carry):
                t0, t1, thr, cnt = carry
                v = bmax_v[pl.ds(j * L, L)]
                m = v > thr

                def take(args):
                    t0, t1, thr_, cnt = args
                    plsc.store_compressed(cand_v.at[pl.ds(cnt, L)], v, mask=m)
                    cnt = cnt + _popcount(m)

                    def do_flush(args):
                        t0, t1, cnt = args
                        t0, t1 = flush_vals(t0, t1, cnt)
                        return t0, t1, jnp.min(t1), jnp.int32(0)

                    return lax.cond(cnt >= CAP_FLUSH, do_flush,
                                    lambda a: (a[0], a[1], thr_, a[2]),
                                    (t0, t1, cnt))

                return lax.cond(jnp.any(m), take, lambda a: a,
                                (t0, t1, thr, cnt))

            t0, t1, thr, cnt = lax.fori_loop(
                2, nvec, body_a, (t0, t1, thr, jnp.int32(0)))
            t0, t1 = flush_vals(t0, t1, cnt)
            tau = jnp.min(t1)

            def body_b(j, cntb):
                v = bmax_v[pl.ds(j * L, L)]
                m = v >= tau

                def take(cntb):
                    ids = j * L + _iota16()
                    plsc.store_compressed(bid_v.at[pl.ds(cntb, L)], ids,
                                          mask=m)
                    return cntb + _popcount(m)

                return lax.cond(jnp.any(m), take, lambda c: c, cntb)

            cntb = lax.fori_loop(0, nvec, body_b, jnp.int32(0))

            tk0 = jnp.full((L,), NEGF)
            ti0 = jnp.full((L,), BIGI)
            carry0 = (tk0, ti0, tk0, ti0, jnp.int32(0))
            nchunks = (cntb + CH - 1) // CH

            def chunk_body(c, carry):
                base = c * CH
                for jj in range(CH // L):
                    pos = base + jj * L
                    ids = bid_v[pl.ds(pos, L)]
                    ids = jnp.where(pos + _iota16() < cntb, ids, 0)
                    gid_v[pl.ds(jj * L, L)] = ids + q * nb
                pltpu.async_copy(simb_hbm.at[gid_v], blocks_v, sem).wait()
                nblk = jnp.minimum(cntb - base, CH)

                def iblock(i, c2):
                    bidv = plsc.load_gather(
                        bid_v, [jnp.full((L,), base + i, jnp.int32)])
                    for jj in range(GRP // L):
                        v = blocks_v[i, pl.ds(jj * L, L)]
                        m = v >= tau
                        idv = bidv * GRP + (jj * L) + _iota16()

                        def take(args, v=v, m=m, idv=idv):
                            t0, i0, t1, i1, cnt = args
                            plsc.store_compressed(
                                cand_v.at[pl.ds(cnt, L)], v, mask=m)
                            plsc.store_compressed(
                                candi_v.at[pl.ds(cnt, L)], idv, mask=m)
                            cnt = cnt + _popcount(m)

                            def do_flush(args):
                                t0, i0, t1, i1, cnt = args
                                t0, i0, t1, i1 = flush_kv(t0, i0, t1, i1, cnt)
                                return t0, i0, t1, i1, jnp.int32(0)

                            return lax.cond(cnt >= CAP_FLUSH, do_flush,
                                            lambda a: a,
                                            (t0, i0, t1, i1, cnt))

                        c2 = lax.cond(jnp.any(m), take, lambda a: a, c2)
                    return c2

                return lax.fori_loop(0, nblk, iblock, carry)

            t0, i0, t1, i1, cnt = lax.fori_loop(
                0, nchunks, chunk_body, carry0)
            t0, i0, t1, i1 = flush_kv(t0, i0, t1, i1, cnt)

            mx = jnp.max(t0)
            e0 = jnp.exp(t0 - mx)
            e1 = jnp.exp(t1 - mx)
            s = jnp.sum(e0) + jnp.sum(e1)
            w0 = e0 / s
            w1 = e1 / s
            sw = jnp.sum(w0) + jnp.sum(w1) + 1e-8
            w0 = w0 / sw
            w1 = w1 / sw
            idx_v[pl.ds(0, L)] = i0
            idx_v[pl.ds(L, L)] = i1
            w_v[pl.ds(0, L)] = w0
            w_v[pl.ds(L, L)] = w1

            pltpu.async_copy(values_hbm.at[idx_v], rows_v, sem).wait()

            def acc_body(i, accs):
                wspl = plsc.load_gather(
                    w_v, [jnp.full((L,), i, jnp.int32)])
                return tuple(
                    accs[dd] + wspl * rows_v[i, pl.ds(dd * L, L)]
                    for dd in range(d // L))

            accs = lax.fori_loop(
                0, 32, acc_body,
                tuple(jnp.zeros((L,), jnp.float32) for _ in range(d // L)))
            for dd in range(d // L):
                out_v[pl.ds(dd * L, L)] = accs[dd]
            pltpu.sync_copy(out_v, out_hbm.at[q])
            return 0

        lax.fori_loop(0, qpw, per_query, 0)

    return body(simb, bmax, values)


def kernel(queries, keys, values):
    nq, d = queries.shape
    kn = keys.shape[0]
    kpad = ((kn + KT - 1) // KT) * KT
    nb = kpad // GRP
    keys_p = jnp.pad(keys, ((0, kpad - kn), (0, 0)))
    sim, bmax = _tc_sim(queries, keys_p, kn)
    simb = sim.reshape(nq * nb, GRP)
    return _sc_retrieve(simb, bmax, values)


# speedup vs baseline: 5.9599x; 5.9599x over previous
import functools

import jax
import jax.numpy as jnp
from jax import lax
from jax.experimental import pallas as pl
from jax.experimental.pallas import tpu as pltpu
from jax.experimental.pallas import tpu_sc as plsc

L = 16
GRP = 128
KT = 2048
NC, NS = 2, 16
NW = NC * NS

NEG = -1e30
NEGF = -3e38
BIGI = 2**30

CAP_FLUSH = 64
CH = 64


def _tc_body(nk, q_ref, k_ref, sim_ref, bmax_ref):
    j = pl.program_id(0)
    q = q_ref[...]
    qn = q / (jnp.sqrt(jnp.sum(q * q, axis=1, keepdims=True)) + 1e-8)
    k = k_ref[...]
    kn = k / (jnp.sqrt(jnp.sum(k * k, axis=1, keepdims=True)) + 1e-8)
    sim = lax.dot_general(qn, kn, (((1,), (1,)), ((), ())),
                          preferred_element_type=jnp.float32)
    col = j * KT + lax.broadcasted_iota(jnp.int32, sim.shape, 1)
    sim = jnp.where(col < nk, sim, NEG)
    sim_ref[...] = sim
    bmax_ref[0] = jnp.max(
        sim.reshape(sim.shape[0], KT // GRP, GRP), axis=2)


def _tc_sim(queries, keys_p, nk):
    nq, d = queries.shape
    kpad = keys_p.shape[0]
    grid = kpad // KT
    return pl.pallas_call(
        functools.partial(_tc_body, nk),
        grid=(grid,),
        in_specs=[
            pl.BlockSpec((nq, d), lambda j: (0, 0)),
            pl.BlockSpec((KT, d), lambda j: (j, 0)),
        ],
        out_specs=[
            pl.BlockSpec((nq, KT), lambda j: (0, j)),
            pl.BlockSpec((1, nq, KT // GRP), lambda j: (j, 0, 0)),
        ],
        out_shape=[
            jax.ShapeDtypeStruct((nq, kpad), jnp.float32),
            jax.ShapeDtypeStruct((grid, nq, KT // GRP), jnp.float32),
        ],
    )(queries, keys_p)


def _iota16():
    return lax.iota(jnp.int32, L)


def _sortd(v):
    return lax.rev(lax.sort(v), (0,))


def _sortd_kv(k, v):
    ks, vs = plsc.sort_key_val(k, v, descending=True)
    return ks, vs


def _merge_desc(a, b):
    br = lax.rev(b, (0,))
    m = a >= br
    hi = jnp.where(m, a, br)
    lo = jnp.where(m, br, a)
    return _sortd(hi), _sortd(lo)


def _merge32(c, t0, t1):
    hi, lo = _merge_desc(c, t1)
    t0n, b = _merge_desc(hi, t0)
    t1n, _ = _merge_desc(b, lo)
    return t0n, t1n


def _merge_desc_kv(ak, av, bk, bv):
    brk = lax.rev(bk, (0,))
    brv = lax.rev(bv, (0,))
    m = (ak > brk) | ((ak == brk) & (av < brv))
    hik = jnp.where(m, ak, brk)
    hiv = jnp.where(m, av, brv)
    lok = jnp.where(m, brk, ak)
    lov = jnp.where(m, brv, av)
    hik, hiv = _sortd_kv(hik, hiv)
    lok, lov = _sortd_kv(lok, lov)
    return hik, hiv, lok, lov


def _merge32_kv(ck, cv, t0, i0, t1, i1):
    hik, hiv, lok, lov = _merge_desc_kv(ck, cv, t1, i1)
    t0n, i0n, bk, bv = _merge_desc_kv(hik, hiv, t0, i0)
    t1n, i1n, _, _ = _merge_desc_kv(bk, bv, lok, lov)
    return t0n, i0n, t1n, i1n


def _popcount(m):
    return jnp.max(plsc.all_reduce_population_count(m))


def _sc_retrieve(simb, bmax, values):
    nq, nb = bmax.shape
    kn, d = values.shape
    nvec = nb // L
    qpw = nq // NW
    mesh = plsc.VectorSubcoreMesh(core_axis_name="c", subcore_axis_name="s",
                                  num_cores=NC, num_subcores=NS)

    @functools.partial(
        pl.kernel,
        out_type=jax.ShapeDtypeStruct((nq, d), jnp.float32),
        mesh=mesh,
        compiler_params=pltpu.CompilerParams(needs_layout_passes=False),
        scratch_types=[
            pltpu.VMEM((nb,), jnp.float32),
            pltpu.VMEM((nb + CH,), jnp.int32),
            pltpu.VMEM((96,), jnp.float32),
            pltpu.VMEM((96,), jnp.int32),
            pltpu.VMEM((CH,), jnp.int32),
            pltpu.VMEM((CH, GRP), jnp.float32),
            pltpu.VMEM((32,), jnp.int32),
            pltpu.VMEM((32,), jnp.float32),
            pltpu.VMEM((32, d), jnp.float32),
            pltpu.VMEM((d,), jnp.float32),
            pltpu.SemaphoreType.DMA,
        ],
    )
    def body(simb_hbm, bmax_hbm, values_hbm, out_hbm,
             bmax_v, bid_v, cand_v, candi_v, gid_v, blocks_v,
             idx_v, w_v, rows_v, out_v, sem):
        wid = lax.axis_index("c") * NS + lax.axis_index("s")

        def flush_vals(t0, t1, cnt):
            nch = (cnt + L - 1) // L

            def fb(ci, c2):
                t0, t1 = c2
                v = cand_v[pl.ds(ci * L, L)]
                v = jnp.where(_iota16() < (cnt - ci * L), v, NEGF)
                return _merge32(_sortd(v), t0, t1)

            return lax.fori_loop(0, nch, fb, (t0, t1))

        def flush_kv(t0, i0, t1, i1, cnt):
            nch = (cnt + L - 1) // L

            def fb(ci, c2):
                t0, i0, t1, i1 = c2
                v = cand_v[pl.ds(ci * L, L)]
                iv = candi_v[pl.ds(ci * L, L)]
                valid = _iota16() < (cnt - ci * L)
                v = jnp.where(valid, v, NEGF)
                iv = jnp.where(valid, iv, BIGI)
                ck, cv = _sortd_kv(v, iv)
                return _merge32_kv(ck, cv, t0, i0, t1, i1)

            return lax.fori_loop(0, nch, fb, (t0, i0, t1, i1))

        def per_query(qi, _):
            q = wid * qpw + qi
            pltpu.sync_copy(bmax_hbm.at[q], bmax_v)

            v0 = _sortd(bmax_v[pl.ds(0, L)])
            v1 = _sortd(bmax_v[pl.ds(L, L)])
            t0, t1 = _merge_desc(v0, v1)
            thr = jnp.min(t1)

            def body_a(j, carry):
                t0, t1, thr, cnt = carry
                v = bmax_v[pl.ds(j * L, L)]
                m = v > thr

                def take(args):
                    t0, t1, thr_, cnt = args
                    plsc.store_compressed(cand_v.at[pl.ds(cnt, L)], v, mask=m)
                    cnt = cnt + _popcount(m)

                    def do_flush(args):
                        t0, t1, cnt = args
                        t0, t1 = flush_vals(t0, t1, cnt)
                        return t0, t1, jnp.min(t1), jnp.int32(0)

                    return lax.cond(cnt >= CAP_FLUSH, do_flush,
                                    lambda a: (a[0], a[1], thr_, a[2]),
                                    (t0, t1, cnt))

                return lax.cond(jnp.any(m), take, lambda a: a,
                                (t0, t1, thr, cnt))

            t0, t1, thr, cnt = lax.fori_loop(
                2, nvec, body_a, (t0, t1, thr, jnp.int32(0)))
            t0, t1 = flush_vals(t0, t1, cnt)
            tau = jnp.min(t1)

            def body_b(j, cntb):
                v = bmax_v[pl.ds(j * L, L)]
                m = v >= tau

                def take(cntb):
                    ids = j * L + _iota16()
                    plsc.store_compressed(bid_v.at[pl.ds(cntb, L)], ids,
                                          mask=m)
                    return cntb + _popcount(m)

                return lax.cond(jnp.any(m), take, lambda c: c, cntb)

            cntb = lax.fori_loop(0, nvec, body_b, jnp.int32(0))

            tk0 = jnp.full((L,), NEGF, jnp.float32)
            ti0 = jnp.full((L,), BIGI, jnp.int32)
            carry0 = (tk0, ti0, tk0, ti0, jnp.int32(0))
            nchunks = (cntb + CH - 1) // CH

            def chunk_body(c, carry):
                base = c * CH
                for jj in range(CH // L):
                    pos = base + jj * L
                    ids = bid_v[pl.ds(pos, L)]
                    ids = jnp.where(pos + _iota16() < cntb, ids, 0)
                    gid_v[pl.ds(jj * L, L)] = ids + q * nb
                pltpu.async_copy(simb_hbm.at[gid_v], blocks_v, sem).wait()
                nblk = jnp.minimum(cntb - base, CH)

                def iblock(i, c2):
                    bidv = plsc.load_gather(
                        bid_v, [jnp.full((L,), base + i, jnp.int32)])
                    for jj in range(GRP // L):
                        v = blocks_v[i, pl.ds(jj * L, L)]
                        m = v >= tau
                        idv = bidv * GRP + (jj * L) + _iota16()

                        def take(args, v=v, m=m, idv=idv):
                            t0, i0, t1, i1, cnt = args
                            plsc.store_compressed(
                                cand_v.at[pl.ds(cnt, L)], v, mask=m)
                            plsc.store_compressed(
                                candi_v.at[pl.ds(cnt, L)], idv, mask=m)
                            cnt = cnt + _popcount(m)

                            def do_flush(args):
                                t0, i0, t1, i1, cnt = args
                                t0, i0, t1, i1 = flush_kv(t0, i0, t1, i1, cnt)
                                return t0, i0, t1, i1, jnp.int32(0)

                            return lax.cond(cnt >= CAP_FLUSH, do_flush,
                                            lambda a: a,
                                            (t0, i0, t1, i1, cnt))

                        c2 = lax.cond(jnp.any(m), take, lambda a: a, c2)
                    return c2

                return lax.fori_loop(0, nblk, iblock, carry)

            t0, i0, t1, i1, cnt = lax.fori_loop(
                0, nchunks, chunk_body, carry0)
            t0, i0, t1, i1 = flush_kv(t0, i0, t1, i1, cnt)

            mx = jnp.max(t0)
            e0 = jnp.exp(t0 - mx)
            e1 = jnp.exp(t1 - mx)
            s = jnp.sum(e0) + jnp.sum(e1)
            w0 = e0 / s
            w1 = e1 / s
            sw = jnp.sum(w0) + jnp.sum(w1) + 1e-8
            w0 = w0 / sw
            w1 = w1 / sw
            idx_v[pl.ds(0, L)] = i0
            idx_v[pl.ds(L, L)] = i1
            w_v[pl.ds(0, L)] = w0
            w_v[pl.ds(L, L)] = w1

            pltpu.async_copy(values_hbm.at[idx_v], rows_v, sem).wait()

            def acc_body(i, accs):
                wspl = plsc.load_gather(
                    w_v, [jnp.full((L,), i, jnp.int32)])
                return tuple(
                    accs[dd] + wspl * rows_v[i, pl.ds(dd * L, L)]
                    for dd in range(d // L))

            accs = lax.fori_loop(
                0, 32, acc_body,
                tuple(jnp.zeros((L,), jnp.float32) for _ in range(d // L)))
            for dd in range(d // L):
                out_v[pl.ds(dd * L, L)] = accs[dd]
            pltpu.sync_copy(out_v, out_hbm.at[q])
            return 0

        lax.fori_loop(0, qpw, per_query, 0)

    return body(simb, bmax, values)


def kernel(queries, keys, values):
    nq, d = queries.shape
    kn = keys.shape[0]
    kpad = ((kn + KT - 1) // KT) * KT
    nb = kpad // GRP
    keys_p = jnp.pad(keys, ((0, kpad - kn), (0, 0)))
    sim, bmax3 = _tc_sim(queries, keys_p, kn)
    bmax = bmax3.transpose(1, 0, 2).reshape(nq, nb)
    simb = sim.reshape(nq * nb, GRP)
    return _sc_retrieve(simb, bmax, values)


# speedup vs baseline: 6.2017x; 1.0406x over previous
import functools

import jax
import jax.numpy as jnp
from jax import lax
from jax.experimental import pallas as pl
from jax.experimental.pallas import tpu as pltpu
from jax.experimental.pallas import tpu_sc as plsc

L = 16
GRP = 128
KT = 2048
NC, NS = 2, 16
NW = NC * NS

NEG = -1e30
NEGF = -3e38
BIGI = 2**30

CAP_FLUSH = 64
CH = 64


def _tc_body(nk, q_ref, k_ref, sim_ref, bmax_ref):
    j = pl.program_id(0)
    q = q_ref[...]
    qn = q / (jnp.sqrt(jnp.sum(q * q, axis=1, keepdims=True)) + 1e-8)
    k = k_ref[...]
    kn = k / (jnp.sqrt(jnp.sum(k * k, axis=1, keepdims=True)) + 1e-8)
    sim = lax.dot_general(qn, kn, (((1,), (1,)), ((), ())),
                          preferred_element_type=jnp.float32)
    sim_ref[...] = sim

    def bmax_of(s):
        return jnp.max(s.reshape(s.shape[0], KT // GRP, GRP), axis=2)

    last = pl.num_programs(0) - 1

    @pl.when(j < last)
    def _():
        bmax_ref[0] = bmax_of(sim)

    @pl.when(j == last)
    def _():
        col = j * KT + lax.broadcasted_iota(jnp.int32, sim.shape, 1)
        bmax_ref[0] = bmax_of(jnp.where(col < nk, sim, NEG))


def _tc_sim(queries, keys_p, nk):
    nq, d = queries.shape
    kpad = keys_p.shape[0]
    grid = kpad // KT
    return pl.pallas_call(
        functools.partial(_tc_body, nk),
        grid=(grid,),
        in_specs=[
            pl.BlockSpec((nq, d), lambda j: (0, 0)),
            pl.BlockSpec((KT, d), lambda j: (j, 0)),
        ],
        out_specs=[
            pl.BlockSpec((nq, KT), lambda j: (0, j)),
            pl.BlockSpec((1, nq, KT // GRP), lambda j: (j, 0, 0)),
        ],
        out_shape=[
            jax.ShapeDtypeStruct((nq, kpad), jnp.float32),
            jax.ShapeDtypeStruct((grid, nq, KT // GRP), jnp.float32),
        ],
    )(queries, keys_p)


def _iota16():
    return lax.iota(jnp.int32, L)


def _sortd(v):
    return lax.rev(lax.sort(v), (0,))


def _sortd_kv(k, v):
    ks, vs = plsc.sort_key_val(k, v, descending=True)
    return ks, vs


def _merge_desc(a, b):
    br = lax.rev(b, (0,))
    m = a >= br
    hi = jnp.where(m, a, br)
    lo = jnp.where(m, br, a)
    return _sortd(hi), _sortd(lo)


def _merge32(c, t0, t1):
    hi, lo = _merge_desc(c, t1)
    t0n, b = _merge_desc(hi, t0)
    t1n, _ = _merge_desc(b, lo)
    return t0n, t1n


def _merge_desc_kv(ak, av, bk, bv):
    brk = lax.rev(bk, (0,))
    brv = lax.rev(bv, (0,))
    m = (ak > brk) | ((ak == brk) & (av < brv))
    hik = jnp.where(m, ak, brk)
    hiv = jnp.where(m, av, brv)
    lok = jnp.where(m, brk, ak)
    lov = jnp.where(m, brv, av)
    hik, hiv = _sortd_kv(hik, hiv)
    lok, lov = _sortd_kv(lok, lov)
    return hik, hiv, lok, lov


def _merge32_kv(ck, cv, t0, i0, t1, i1):
    hik, hiv, lok, lov = _merge_desc_kv(ck, cv, t1, i1)
    t0n, i0n, bk, bv = _merge_desc_kv(hik, hiv, t0, i0)
    t1n, i1n, _, _ = _merge_desc_kv(bk, bv, lok, lov)
    return t0n, i0n, t1n, i1n


def _popcount(m):
    return jnp.max(plsc.all_reduce_population_count(m))


def _sc_retrieve(simb, bmax, values):
    nq, nb = bmax.shape
    kn, d = values.shape
    nvec = nb // L
    qpw = nq // NW
    mesh = plsc.VectorSubcoreMesh(core_axis_name="c", subcore_axis_name="s",
                                  num_cores=NC, num_subcores=NS)

    @functools.partial(
        pl.kernel,
        out_type=jax.ShapeDtypeStruct((nq, d), jnp.float32),
        mesh=mesh,
        compiler_params=pltpu.CompilerParams(needs_layout_passes=False),
        scratch_types=[
            pltpu.VMEM((nb,), jnp.float32),
            pltpu.VMEM((nb + CH,), jnp.int32),
            pltpu.VMEM((96,), jnp.float32),
            pltpu.VMEM((96,), jnp.int32),
            pltpu.VMEM((CH,), jnp.int32),
            pltpu.VMEM((CH, GRP), jnp.float32),
            pltpu.VMEM((nq // NW * 32,), jnp.int32),
            pltpu.VMEM((nq // NW * 32,), jnp.float32),
            pltpu.VMEM((4 * 32, d), jnp.float32),
            pltpu.VMEM((4, d), jnp.float32),
            pltpu.SemaphoreType.DMA,
        ],
    )
    def body(simb_hbm, bmax_hbm, values_hbm, out_hbm,
             bmax_v, bid_v, cand_v, candi_v, gid_v, blocks_v,
             idx_v, w_v, rows_v, out_v, sem):
        wid = lax.axis_index("c") * NS + lax.axis_index("s")

        def flush_vals(t0, t1, cnt):
            nch = (cnt + L - 1) // L

            def fb(ci, c2):
                t0, t1 = c2
                v = cand_v[pl.ds(ci * L, L)]
                v = jnp.where(_iota16() < (cnt - ci * L), v, NEGF)
                return _merge32(_sortd(v), t0, t1)

            return lax.fori_loop(0, nch, fb, (t0, t1))

        def flush_kv(t0, i0, t1, i1, cnt):
            nch = (cnt + L - 1) // L

            def fb(ci, c2):
                t0, i0, t1, i1 = c2
                v = cand_v[pl.ds(ci * L, L)]
                iv = candi_v[pl.ds(ci * L, L)]
                valid = _iota16() < (cnt - ci * L)
                v = jnp.where(valid, v, NEGF)
                iv = jnp.where(valid, iv, BIGI)
                ck, cv = _sortd_kv(v, iv)
                return _merge32_kv(ck, cv, t0, i0, t1, i1)

            return lax.fori_loop(0, nch, fb, (t0, i0, t1, i1))

        def per_query(qi, _):
            q = wid * qpw + qi
            pltpu.sync_copy(bmax_hbm.at[q], bmax_v)

            v0 = _sortd(bmax_v[pl.ds(0, L)])
            v1 = _sortd(bmax_v[pl.ds(L, L)])
            t0, t1 = _merge_desc(v0, v1)
            thr = jnp.min(t1)

            def body_a(j, carry):
                t0, t1, thr, cnt = carry
                v = bmax_v[pl.ds(j * L, L)]
                m = v > thr

                def take(args):
                    t0, t1, thr_, cnt = args
                    plsc.store_compressed(cand_v.at[pl.ds(cnt, L)], v, mask=m)
                    cnt = cnt + _popcount(m)

                    def do_flush(args):
                        t0, t1, cnt = args
                        t0, t1 = flush_vals(t0, t1, cnt)
                        return t0, t1, jnp.min(t1), jnp.int32(0)

                    return lax.cond(cnt >= CAP_FLUSH, do_flush,
                                    lambda a: (a[0], a[1], thr_, a[2]),
                                    (t0, t1, cnt))

                return lax.cond(jnp.any(m), take, lambda a: a,
                                (t0, t1, thr, cnt))

            t0, t1, thr, cnt = lax.fori_loop(
                2, nvec, body_a, (t0, t1, thr, jnp.int32(0)))
            t0, t1 = flush_vals(t0, t1, cnt)
            tau = jnp.min(t1)

            def body_b(j, cntb):
                v = bmax_v[pl.ds(j * L, L)]
                m = v >= tau

                def take(cntb):
                    ids = j * L + _iota16()
                    plsc.store_compressed(bid_v.at[pl.ds(cntb, L)], ids,
                                          mask=m)
                    return cntb + _popcount(m)

                return lax.cond(jnp.any(m), take, lambda c: c, cntb)

            cntb = lax.fori_loop(0, nvec, body_b, jnp.int32(0))

            tk0 = jnp.full((L,), NEGF, jnp.float32)
            ti0 = jnp.full((L,), BIGI, jnp.int32)
            carry0 = (tk0, ti0, tk0, ti0, jnp.int32(0))
            nchunks = (cntb + CH - 1) // CH

            def chunk_body(c, carry):
                base = c * CH
                for jj in range(CH // L):
                    pos = base + jj * L
                    ids = bid_v[pl.ds(pos, L)]
                    ids = jnp.where(pos + _iota16() < cntb, ids, 0)
                    gid_v[pl.ds(jj * L, L)] = ids + q * nb
                pltpu.async_copy(simb_hbm.at[gid_v], blocks_v, sem).wait()
                nblk = jnp.minimum(cntb - base, CH)

                def iblock(i, c2):
                    bidv = plsc.load_gather(
                        bid_v, [jnp.full((L,), base + i, jnp.int32)])
                    for jj in range(GRP // L):
                        v = blocks_v[i, pl.ds(jj * L, L)]
                        idv = bidv * GRP + (jj * L) + _iota16()
                        m = (v >= tau) & (idv < kn)

                        def take(args, v=v, m=m, idv=idv):
                            t0, i0, t1, i1, cnt = args
                            plsc.store_compressed(
                                cand_v.at[pl.ds(cnt, L)], v, mask=m)
                            plsc.store_compressed(
                                candi_v.at[pl.ds(cnt, L)], idv, mask=m)
                            cnt = cnt + _popcount(m)

                            def do_flush(args):
                                t0, i0, t1, i1, cnt = args
                                t0, i0, t1, i1 = flush_kv(t0, i0, t1, i1, cnt)
                                return t0, i0, t1, i1, jnp.int32(0)

                            return lax.cond(cnt >= CAP_FLUSH, do_flush,
                                            lambda a: a,
                                            (t0, i0, t1, i1, cnt))

                        c2 = lax.cond(jnp.any(m), take, lambda a: a, c2)
                    return c2

                return lax.fori_loop(0, nblk, iblock, carry)

            t0, i0, t1, i1, cnt = lax.fori_loop(
                0, nchunks, chunk_body, carry0)
            t0, i0, t1, i1 = flush_kv(t0, i0, t1, i1, cnt)

            mx = jnp.max(t0)
            e0 = jnp.exp(t0 - mx)
            e1 = jnp.exp(t1 - mx)
            s = jnp.sum(e0) + jnp.sum(e1)
            w0 = e0 / s
            w1 = e1 / s
            sw = jnp.sum(w0) + jnp.sum(w1) + 1e-8
            w0 = w0 / sw
            w1 = w1 / sw
            idx_v[pl.ds(qi * 32, L)] = i0
            idx_v[pl.ds(qi * 32 + L, L)] = i1
            w_v[pl.ds(qi * 32, L)] = w0
            w_v[pl.ds(qi * 32 + L, L)] = w1
            return 0

        lax.fori_loop(0, qpw, per_query, 0)

        def pass2(g, _):
            qbase = wid * qpw + g * 4
            pltpu.async_copy(
                values_hbm.at[idx_v.at[pl.ds(g * 128, 128)]], rows_v,
                sem).wait()
            for qq in range(4):
                def acc_body(i, accs, qq=qq):
                    wspl = plsc.load_gather(
                        w_v, [jnp.full((L,), g * 128 + qq * 32 + i,
                                       jnp.int32)])
                    return tuple(
                        accs[dd] + wspl * rows_v[qq * 32 + i,
                                                 pl.ds(dd * L, L)]
                        for dd in range(d // L))

                accs = lax.fori_loop(
                    0, 32, acc_body,
                    tuple(jnp.zeros((L,), jnp.float32)
                          for _ in range(d // L)))
                for dd in range(d // L):
                    out_v[qq, pl.ds(dd * L, L)] = accs[dd]
            pltpu.sync_copy(out_v, out_hbm.at[pl.ds(qbase, 4)])
            return 0

        lax.fori_loop(0, qpw // 4, pass2, 0)

    return body(simb, bmax, values)


def kernel(queries, keys, values):
    nq, d = queries.shape
    kn = keys.shape[0]
    kpad = ((kn + KT - 1) // KT) * KT
    nb = kpad // GRP
    keys_p = jnp.pad(keys, ((0, kpad - kn), (0, 0)))
    sim, bmax3 = _tc_sim(queries, keys_p, kn)
    bmax = bmax3.transpose(1, 0, 2).reshape(nq, nb)
    simb = sim.reshape(nq * nb, GRP)
    return _sc_retrieve(simb, bmax, values)


# speedup vs baseline: 8.0595x; 1.2996x over previous
import functools

import jax
import jax.numpy as jnp
from jax import lax
from jax.experimental import pallas as pl
from jax.experimental.pallas import tpu as pltpu
from jax.experimental.pallas import tpu_sc as plsc

L = 16
GRP = 128
KT = 2048
NC, NS = 2, 16
NW = NC * NS

NEG = -1e30
NEGF = -3e38
BIGI = 2**30

CAP_FLUSH = 64
CH = 48
NSPLIT = 4


def _tc_body(nk, q_ref, k_ref, sim_ref, bmax_ref):
    j = pl.program_id(0)
    q = q_ref[...]
    qn = q / (jnp.sqrt(jnp.sum(q * q, axis=1, keepdims=True)) + 1e-8)
    k = k_ref[...]
    kn = k / (jnp.sqrt(jnp.sum(k * k, axis=1, keepdims=True)) + 1e-8)
    sim = lax.dot_general(qn, kn, (((1,), (1,)), ((), ())),
                          preferred_element_type=jnp.float32)
    sim_ref[...] = sim

    def bmax_of(s):
        return jnp.max(s.reshape(s.shape[0], KT // GRP, GRP), axis=2)

    last = pl.num_programs(0) - 1

    @pl.when(j < last)
    def _():
        bmax_ref[0] = bmax_of(sim)

    @pl.when(j == last)
    def _():
        col = j * KT + lax.broadcasted_iota(jnp.int32, sim.shape, 1)
        bmax_ref[0] = bmax_of(jnp.where(col < nk, sim, NEG))


def _tc_sim(queries, keys_p, nk):
    nq, d = queries.shape
    kpad = keys_p.shape[0]
    grid = kpad // KT
    return pl.pallas_call(
        functools.partial(_tc_body, nk),
        grid=(grid,),
        in_specs=[
            pl.BlockSpec((nq, d), lambda j: (0, 0)),
            pl.BlockSpec((KT, d), lambda j: (j, 0)),
        ],
        out_specs=[
            pl.BlockSpec((nq, KT), lambda j: (0, j)),
            pl.BlockSpec((1, nq, KT // GRP), lambda j: (j, 0, 0)),
        ],
        out_shape=[
            jax.ShapeDtypeStruct((nq, kpad), jnp.float32),
            jax.ShapeDtypeStruct((grid, nq, KT // GRP), jnp.float32),
        ],
    )(queries, keys_p)


def _iota16():
    return lax.iota(jnp.int32, L)


def _sortd(v):
    return lax.rev(lax.sort(v), (0,))


def _sortd_kv(k, v):
    ks, vs = plsc.sort_key_val(k, v, descending=True)
    return ks, vs


def _merge_desc(a, b):
    br = lax.rev(b, (0,))
    m = a >= br
    hi = jnp.where(m, a, br)
    lo = jnp.where(m, br, a)
    return _sortd(hi), _sortd(lo)


def _merge32(c, t0, t1):
    hi, lo = _merge_desc(c, t1)
    t0n, b = _merge_desc(hi, t0)
    t1n, _ = _merge_desc(b, lo)
    return t0n, t1n


def _merge_desc_kv(ak, av, bk, bv):
    brk = lax.rev(bk, (0,))
    brv = lax.rev(bv, (0,))
    m = (ak > brk) | ((ak == brk) & (av < brv))
    hik = jnp.where(m, ak, brk)
    hiv = jnp.where(m, av, brv)
    lok = jnp.where(m, brk, ak)
    lov = jnp.where(m, brv, av)
    hik, hiv = _sortd_kv(hik, hiv)
    lok, lov = _sortd_kv(lok, lov)
    return hik, hiv, lok, lov


def _merge32_kv(ck, cv, t0, i0, t1, i1):
    hik, hiv, lok, lov = _merge_desc_kv(ck, cv, t1, i1)
    t0n, i0n, bk, bv = _merge_desc_kv(hik, hiv, t0, i0)
    t1n, i1n, _, _ = _merge_desc_kv(bk, bv, lok, lov)
    return t0n, i0n, t1n, i1n


def _popcount(m):
    return jnp.max(plsc.all_reduce_population_count(m))


def _sc_retrieve(simb, bmax, values):
    nq, nb = bmax.shape
    kn, d = values.shape
    nvec = nb // L
    qpw = nq // NW
    mesh = plsc.VectorSubcoreMesh(core_axis_name="c", subcore_axis_name="s",
                                  num_cores=NC, num_subcores=NS)

    @functools.partial(
        pl.kernel,
        out_type=jax.ShapeDtypeStruct((nq, d), jnp.float32),
        mesh=mesh,
        compiler_params=pltpu.CompilerParams(needs_layout_passes=False),
        scratch_types=[
            pltpu.VMEM((nb,), jnp.float32),
            pltpu.VMEM((nb + CH,), jnp.int32),
            pltpu.VMEM((96,), jnp.float32),
            pltpu.VMEM((96,), jnp.int32),
            pltpu.VMEM((CH,), jnp.int32),
            pltpu.VMEM((CH, GRP), jnp.float32),
            pltpu.VMEM((nq // NW * 32,), jnp.int32),
            pltpu.VMEM((nq // NW * 32,), jnp.float32),
            pltpu.VMEM((4 * 32, d), jnp.float32),
            pltpu.VMEM((4, d), jnp.float32),
            pltpu.SemaphoreType.DMA,
        ],
    )
    def body(simb_hbm, bmax_hbm, values_hbm, out_hbm,
             bmax_v, bid_v, cand_v, candi_v, gid_v, blocks_v,
             idx_v, w_v, rows_v, out_v, sem):
        wid = lax.axis_index("c") * NS + lax.axis_index("s")

        def flush_vals(t0, t1, cnt):
            nch = (cnt + L - 1) // L

            def fb(ci, c2):
                t0, t1 = c2
                v = cand_v[pl.ds(ci * L, L)]
                v = jnp.where(_iota16() < (cnt - ci * L), v, NEGF)
                return _merge32(_sortd(v), t0, t1)

            return lax.fori_loop(0, nch, fb, (t0, t1))

        def flush_kv(t0, i0, t1, i1, cnt):
            nch = (cnt + L - 1) // L

            def fb(ci, c2):
                t0, i0, t1, i1 = c2
                v = cand_v[pl.ds(ci * L, L)]
                iv = candi_v[pl.ds(ci * L, L)]
                valid = _iota16() < (cnt - ci * L)
                v = jnp.where(valid, v, NEGF)
                iv = jnp.where(valid, iv, BIGI)
                ck, cv = _sortd_kv(v, iv)
                return _merge32_kv(ck, cv, t0, i0, t1, i1)

            return lax.fori_loop(0, nch, fb, (t0, i0, t1, i1))

        def per_query(qi, _):
            q = wid * qpw + qi
            pltpu.sync_copy(bmax_hbm.at[q], bmax_v)

            v0 = _sortd(bmax_v[pl.ds(0, L)])
            v1 = _sortd(bmax_v[pl.ds(L, L)])
            t0, t1 = _merge_desc(v0, v1)
            thr = jnp.min(t1)

            def body_a(j, carry):
                t0, t1, thr, cnt = carry
                v = bmax_v[pl.ds(j * L, L)]
                m = v > thr

                def take(args):
                    t0, t1, thr_, cnt = args
                    plsc.store_compressed(cand_v.at[pl.ds(cnt, L)], v, mask=m)
                    cnt = cnt + _popcount(m)

                    def do_flush(args):
                        t0, t1, cnt = args
                        t0, t1 = flush_vals(t0, t1, cnt)
                        return t0, t1, jnp.min(t1), jnp.int32(0)

                    return lax.cond(cnt >= CAP_FLUSH, do_flush,
                                    lambda a: (a[0], a[1], thr_, a[2]),
                                    (t0, t1, cnt))

                return lax.cond(jnp.any(m), take, lambda a: a,
                                (t0, t1, thr, cnt))

            t0, t1, thr, cnt = lax.fori_loop(
                2, nvec, body_a, (t0, t1, thr, jnp.int32(0)))
            t0, t1 = flush_vals(t0, t1, cnt)
            tau = jnp.min(t1)

            def body_b(j, cntb):
                v = bmax_v[pl.ds(j * L, L)]
                m = v >= tau

                def take(cntb):
                    ids = j * L + _iota16()
                    plsc.store_compressed(bid_v.at[pl.ds(cntb, L)], ids,
                                          mask=m)
                    return cntb + _popcount(m)

                return lax.cond(jnp.any(m), take, lambda c: c, cntb)

            cntb = lax.fori_loop(0, nvec, body_b, jnp.int32(0))

            tk0 = jnp.full((L,), NEGF, jnp.float32)
            ti0 = jnp.full((L,), BIGI, jnp.int32)
            carry0 = (tk0, ti0, tk0, ti0, jnp.int32(0))
            nchunks = (cntb + CH - 1) // CH

            def chunk_body(c, carry):
                base = c * CH
                for jj in range(CH // L):
                    pos = base + jj * L
                    ids = bid_v[pl.ds(pos, L)]
                    ids = jnp.where(pos + _iota16() < cntb, ids, 0)
                    gid_v[pl.ds(jj * L, L)] = ids + q * nb
                pltpu.async_copy(simb_hbm.at[gid_v], blocks_v, sem).wait()
                nblk = jnp.minimum(cntb - base, CH)

                def iblock(i, c2):
                    bidv = plsc.load_gather(
                        bid_v, [jnp.full((L,), base + i, jnp.int32)])
                    for jj in range(GRP // L):
                        v = blocks_v[i, pl.ds(jj * L, L)]
                        idv = bidv * GRP + (jj * L) + _iota16()
                        m = (v >= tau) & (idv < kn)

                        def take(args, v=v, m=m, idv=idv):
                            t0, i0, t1, i1, cnt = args
                            plsc.store_compressed(
                                cand_v.at[pl.ds(cnt, L)], v, mask=m)
                            plsc.store_compressed(
                                candi_v.at[pl.ds(cnt, L)], idv, mask=m)
                            cnt = cnt + _popcount(m)

                            def do_flush(args):
                                t0, i0, t1, i1, cnt = args
                                t0, i0, t1, i1 = flush_kv(t0, i0, t1, i1, cnt)
                                return t0, i0, t1, i1, jnp.int32(0)

                            return lax.cond(cnt >= CAP_FLUSH, do_flush,
                                            lambda a: a,
                                            (t0, i0, t1, i1, cnt))

                        c2 = lax.cond(jnp.any(m), take, lambda a: a, c2)
                    return c2

                return lax.fori_loop(0, nblk, iblock, carry)

            t0, i0, t1, i1, cnt = lax.fori_loop(
                0, nchunks, chunk_body, carry0)
            t0, i0, t1, i1 = flush_kv(t0, i0, t1, i1, cnt)

            mx = jnp.max(t0)
            e0 = jnp.exp(t0 - mx)
            e1 = jnp.exp(t1 - mx)
            s = jnp.sum(e0) + jnp.sum(e1)
            w0 = e0 / s
            w1 = e1 / s
            sw = jnp.sum(w0) + jnp.sum(w1) + 1e-8
            w0 = w0 / sw
            w1 = w1 / sw
            idx_v[pl.ds(qi * 32, L)] = i0
            idx_v[pl.ds(qi * 32 + L, L)] = i1
            w_v[pl.ds(qi * 32, L)] = w0
            w_v[pl.ds(qi * 32 + L, L)] = w1
            return 0

        lax.fori_loop(0, qpw, per_query, 0)

        def pass2(g, _):
            qbase = wid * qpw + g * 4
            pltpu.async_copy(
                values_hbm.at[idx_v.at[pl.ds(g * 128, 128)]], rows_v,
                sem).wait()
            for qq in range(4):
                def acc_body(i, accs, qq=qq):
                    wspl = plsc.load_gather(
                        w_v, [jnp.full((L,), g * 128 + qq * 32 + i,
                                       jnp.int32)])
                    return tuple(
                        accs[dd] + wspl * rows_v[qq * 32 + i,
                                                 pl.ds(dd * L, L)]
                        for dd in range(d // L))

                accs = lax.fori_loop(
                    0, 32, acc_body,
                    tuple(jnp.zeros((L,), jnp.float32)
                          for _ in range(d // L)))
                for dd in range(d // L):
                    out_v[qq, pl.ds(dd * L, L)] = accs[dd]
            pltpu.sync_copy(out_v, out_hbm.at[pl.ds(qbase, 4)])
            return 0

        lax.fori_loop(0, qpw // 4, pass2, 0)

    return body(simb, bmax, values)


def kernel(queries, keys, values):
    nq, d = queries.shape
    kn = keys.shape[0]
    kpad = ((kn + KT - 1) // KT) * KT
    nb = kpad // GRP
    keys_p = jnp.pad(keys, ((0, kpad - kn), (0, 0)))
    qc = nq // NSPLIT
    outs = []
    for s in range(NSPLIT):
        qs = jax.lax.slice_in_dim(queries, s * qc, (s + 1) * qc, axis=0)
        sim, bmax3 = _tc_sim(qs, keys_p, kn)
        bmax = bmax3.transpose(1, 0, 2).reshape(qc, nb)
        simb = sim.reshape(qc * nb, GRP)
        outs.append(_sc_retrieve(simb, bmax, values))
    return jnp.concatenate(outs, axis=0)


# speedup vs baseline: 9.0579x; 1.1239x over previous
import functools

import jax
import jax.numpy as jnp
from jax import lax
from jax.experimental import pallas as pl
from jax.experimental.pallas import tpu as pltpu
from jax.experimental.pallas import tpu_sc as plsc

L = 16
GRP = 128
KT = 2048
NC, NS = 2, 16
NW = NC * NS

NEG = -1e30
NEGF = -3e38
BIGI = 2**30

CAP_FLUSH = 64
CH = 48
NSPLIT = 4


def _tc_body(nk, q_ref, k_ref, sim_ref, bmax_ref):
    j = pl.program_id(0)
    q = q_ref[...]
    qn = q / (jnp.sqrt(jnp.sum(q * q, axis=1, keepdims=True)) + 1e-8)
    k = k_ref[...]
    kn = k / (jnp.sqrt(jnp.sum(k * k, axis=1, keepdims=True)) + 1e-8)
    sim = lax.dot_general(qn, kn, (((1,), (1,)), ((), ())),
                          preferred_element_type=jnp.float32)
    sim_ref[...] = sim

    def bmax_of(s):
        return jnp.max(s.reshape(s.shape[0], KT // GRP, GRP), axis=2)

    last = pl.num_programs(0) - 1

    @pl.when(j < last)
    def _():
        bmax_ref[0] = bmax_of(sim)

    @pl.when(j == last)
    def _():
        col = j * KT + lax.broadcasted_iota(jnp.int32, sim.shape, 1)
        bmax_ref[0] = bmax_of(jnp.where(col < nk, sim, NEG))


def _tc_sim(queries, keys_p, nk):
    nq, d = queries.shape
    kpad = keys_p.shape[0]
    grid = kpad // KT
    return pl.pallas_call(
        functools.partial(_tc_body, nk),
        grid=(grid,),
        in_specs=[
            pl.BlockSpec((nq, d), lambda j: (0, 0)),
            pl.BlockSpec((KT, d), lambda j: (j, 0)),
        ],
        out_specs=[
            pl.BlockSpec((nq, KT), lambda j: (0, j)),
            pl.BlockSpec((1, nq, KT // GRP), lambda j: (j, 0, 0)),
        ],
        out_shape=[
            jax.ShapeDtypeStruct((nq, kpad), jnp.float32),
            jax.ShapeDtypeStruct((grid, nq, KT // GRP), jnp.float32),
        ],
    )(queries, keys_p)


def _iota16():
    return lax.iota(jnp.int32, L)


def _sortd(v):
    return lax.rev(lax.sort(v), (0,))


def _sortd_kv(k, v):
    ks, vs = plsc.sort_key_val(k, v, descending=True)
    return ks, vs


def _merge_desc(a, b):
    br = lax.rev(b, (0,))
    m = a >= br
    hi = jnp.where(m, a, br)
    lo = jnp.where(m, br, a)
    return _sortd(hi), _sortd(lo)


def _merge32(c, t0, t1):
    hi, lo = _merge_desc(c, t1)
    t0n, b = _merge_desc(hi, t0)
    t1n, _ = _merge_desc(b, lo)
    return t0n, t1n


def _merge_desc_kv(ak, av, bk, bv):
    brk = lax.rev(bk, (0,))
    brv = lax.rev(bv, (0,))
    m = (ak > brk) | ((ak == brk) & (av < brv))
    hik = jnp.where(m, ak, brk)
    hiv = jnp.where(m, av, brv)
    lok = jnp.where(m, brk, ak)
    lov = jnp.where(m, brv, av)
    hik, hiv = _sortd_kv(hik, hiv)
    lok, lov = _sortd_kv(lok, lov)
    return hik, hiv, lok, lov


def _merge32_kv(ck, cv, t0, i0, t1, i1):
    hik, hiv, lok, lov = _merge_desc_kv(ck, cv, t1, i1)
    t0n, i0n, bk, bv = _merge_desc_kv(hik, hiv, t0, i0)
    t1n, i1n, _, _ = _merge_desc_kv(bk, bv, lok, lov)
    return t0n, i0n, t1n, i1n


def _popcount(m):
    return jnp.max(plsc.all_reduce_population_count(m))


def _sc_retrieve(simb, bmax, values):
    nq, nb = bmax.shape
    kn, d = values.shape
    nvec = nb // L
    qpw = nq // NW
    mesh = plsc.VectorSubcoreMesh(core_axis_name="c", subcore_axis_name="s",
                                  num_cores=NC, num_subcores=NS)

    @functools.partial(
        pl.kernel,
        out_type=jax.ShapeDtypeStruct((nq, d), jnp.float32),
        mesh=mesh,
        compiler_params=pltpu.CompilerParams(needs_layout_passes=False),
        scratch_types=[
            pltpu.VMEM((nb,), jnp.float32),
            pltpu.VMEM((nb + CH,), jnp.int32),
            pltpu.VMEM((((nvec + L - 1) // L) * L,), jnp.float32),
            pltpu.VMEM((nb + L,), jnp.float32),
            pltpu.VMEM((nb + L,), jnp.int32),
            pltpu.VMEM((CH,), jnp.int32),
            pltpu.VMEM((CH, GRP), jnp.float32),
            pltpu.VMEM((nq // NW * 32,), jnp.int32),
            pltpu.VMEM((nq // NW * 32,), jnp.float32),
            pltpu.VMEM((4 * 32, d), jnp.float32),
            pltpu.VMEM((4, d), jnp.float32),
            pltpu.SemaphoreType.DMA,
        ],
    )
    def body(simb_hbm, bmax_hbm, values_hbm, out_hbm,
             bmax_v, bid_v, mx_v, cand_v, candi_v, gid_v, blocks_v,
             idx_v, w_v, rows_v, out_v, sem):
        wid = lax.axis_index("c") * NS + lax.axis_index("s")
        nvec_pad = ((nvec + L - 1) // L) * L
        zeros_i = jnp.zeros((L,), jnp.int32)
        negf_v = jnp.full((L,), NEGF, jnp.float32)
        last_lane = _iota16() == (L - 1)

        for t in range(nvec // L * L, nvec_pad, L):
            mx_v[pl.ds(t, L)] = negf_v

        def capture(dst_v, x, m, cnt_spl):
            pos = cnt_spl + plsc.cumsum(m.astype(jnp.int32)) - 1
            plsc.store_scatter(dst_v, [pos], x, mask=m)
            return cnt_spl + plsc.all_reduce_population_count(m)

        def flush_vals(t0, t1, cnt):
            nch = (cnt + L - 1) // L

            def fb(ci, c2):
                t0, t1 = c2
                v = cand_v[pl.ds(ci * L, L)]
                v = jnp.where(_iota16() < (cnt - ci * L), v, NEGF)
                return _merge32(_sortd(v), t0, t1)

            return lax.fori_loop(0, nch, fb, (t0, t1))

        def flush_kv(t0, i0, t1, i1, cnt):
            nch = (cnt + L - 1) // L

            def fb(ci, c2):
                t0, i0, t1, i1 = c2
                v = cand_v[pl.ds(ci * L, L)]
                iv = candi_v[pl.ds(ci * L, L)]
                valid = _iota16() < (cnt - ci * L)
                v = jnp.where(valid, v, NEGF)
                iv = jnp.where(valid, iv, BIGI)
                ck, cv = _sortd_kv(v, iv)
                return _merge32_kv(ck, cv, t0, i0, t1, i1)

            return lax.fori_loop(0, nch, fb, (t0, i0, t1, i1))

        def per_query(qi, _):
            q = wid * qpw + qi
            pltpu.sync_copy(bmax_hbm.at[q], bmax_v)

            def body_a1(j, _):
                cm = plsc.cummax(bmax_v[pl.ds(j * L, L)])
                plsc.store_scatter(mx_v, [jnp.full((L,), j, jnp.int32)],
                                   cm, mask=last_lane)
                return 0

            lax.fori_loop(0, nvec, body_a1, 0)

            t0 = _sortd(mx_v[pl.ds(0, L)])
            t1 = negf_v
            for t in range(L, nvec_pad, L):
                t0, t1 = _merge32(_sortd(mx_v[pl.ds(t, L)]), t0, t1)
            tau1 = jnp.min(t1)

            def body_a3(j, cnt_spl):
                v = bmax_v[pl.ds(j * L, L)]
                return capture(cand_v, v, v >= tau1, cnt_spl)

            ccount = jnp.max(lax.fori_loop(0, nvec, body_a3, zeros_i))

            t0, t1 = flush_vals(negf_v, negf_v, ccount)
            tau = jnp.min(t1)

            def body_b(j, cb_spl):
                v = bmax_v[pl.ds(j * L, L)]
                ids = j * L + _iota16()
                return capture(bid_v, ids, v >= tau, cb_spl)

            cntb = jnp.max(lax.fori_loop(0, nvec, body_b, zeros_i))

            tk0 = jnp.full((L,), NEGF, jnp.float32)
            ti0 = jnp.full((L,), BIGI, jnp.int32)
            carry0 = (tk0, ti0, tk0, ti0, jnp.int32(0))
            nchunks = (cntb + CH - 1) // CH

            def chunk_body(c, carry):
                base = c * CH
                for jj in range(CH // L):
                    pos = base + jj * L
                    ids = bid_v[pl.ds(pos, L)]
                    ids = jnp.where(pos + _iota16() < cntb, ids, 0)
                    gid_v[pl.ds(jj * L, L)] = ids + q * nb
                pltpu.async_copy(simb_hbm.at[gid_v], blocks_v, sem).wait()
                nblk = jnp.minimum(cntb - base, CH)

                def iblock(i, c2):
                    t0, i0, t1, i1, cnt = c2
                    bidv = plsc.load_gather(
                        bid_v, [jnp.full((L,), base + i, jnp.int32)])
                    cnt_spl = jnp.full((L,), cnt, jnp.int32)
                    for jj in range(GRP // L):
                        v = blocks_v[i, pl.ds(jj * L, L)]
                        idv = bidv * GRP + (jj * L) + _iota16()
                        m = (v >= tau) & (idv < kn)
                        pos = cnt_spl + plsc.cumsum(m.astype(jnp.int32)) - 1
                        plsc.store_scatter(cand_v, [pos], v, mask=m)
                        plsc.store_scatter(candi_v, [pos], idv, mask=m)
                        cnt_spl = (cnt_spl
                                   + plsc.all_reduce_population_count(m))
                    cnt = jnp.max(cnt_spl)

                    def do_flush(args):
                        t0, i0, t1, i1, cnt = args
                        t0, i0, t1, i1 = flush_kv(t0, i0, t1, i1, cnt)
                        return t0, i0, t1, i1, jnp.int32(0)

                    return lax.cond(cnt >= CAP_FLUSH, do_flush,
                                    lambda a: a, (t0, i0, t1, i1, cnt))

                return lax.fori_loop(0, nblk, iblock, carry)

            t0, i0, t1, i1, cnt = lax.fori_loop(
                0, nchunks, chunk_body, carry0)
            t0, i0, t1, i1 = flush_kv(t0, i0, t1, i1, cnt)

            mx = jnp.max(t0)
            e0 = jnp.exp(t0 - mx)
            e1 = jnp.exp(t1 - mx)
            s = jnp.sum(e0) + jnp.sum(e1)
            w0 = e0 / s
            w1 = e1 / s
            sw = jnp.sum(w0) + jnp.sum(w1) + 1e-8
            w0 = w0 / sw
            w1 = w1 / sw
            idx_v[pl.ds(qi * 32, L)] = i0
            idx_v[pl.ds(qi * 32 + L, L)] = i1
            w_v[pl.ds(qi * 32, L)] = w0
            w_v[pl.ds(qi * 32 + L, L)] = w1
            return 0

        lax.fori_loop(0, qpw, per_query, 0)

        def pass2(g, _):
            qbase = wid * qpw + g * 4
            pltpu.async_copy(
                values_hbm.at[idx_v.at[pl.ds(g * 128, 128)]], rows_v,
                sem).wait()
            for qq in range(4):
                def acc_body(i, accs, qq=qq):
                    wspl = plsc.load_gather(
                        w_v, [jnp.full((L,), g * 128 + qq * 32 + i,
                                       jnp.int32)])
                    return tuple(
                        accs[dd] + wspl * rows_v[qq * 32 + i,
                                                 pl.ds(dd * L, L)]
                        for dd in range(d // L))

                accs = lax.fori_loop(
                    0, 32, acc_body,
                    tuple(jnp.zeros((L,), jnp.float32)
                          for _ in range(d // L)))
                for dd in range(d // L):
                    out_v[qq, pl.ds(dd * L, L)] = accs[dd]
            pltpu.sync_copy(out_v, out_hbm.at[pl.ds(qbase, 4)])
            return 0

        lax.fori_loop(0, qpw // 4, pass2, 0)

    return body(simb, bmax, values)


def kernel(queries, keys, values):
    nq, d = queries.shape
    kn = keys.shape[0]
    kpad = ((kn + KT - 1) // KT) * KT
    nb = kpad // GRP
    keys_p = jnp.pad(keys, ((0, kpad - kn), (0, 0)))
    qc = nq // NSPLIT
    outs = []
    for s in range(NSPLIT):
        qs = jax.lax.slice_in_dim(queries, s * qc, (s + 1) * qc, axis=0)
        sim, bmax3 = _tc_sim(qs, keys_p, kn)
        bmax = bmax3.transpose(1, 0, 2).reshape(qc, nb)
        simb = sim.reshape(qc * nb, GRP)
        outs.append(_sc_retrieve(simb, bmax, values))
    return jnp.concatenate(outs, axis=0)
